# Initial kernel scaffold; baseline (speedup 1.0000x reference)
import jax
import jax.numpy as jnp
from jax.experimental import pallas as pl


def kernel(x, edge_index, edge_attr, pos, params):
    raise NotImplementedError("write your pallas kernel here")


# speedup vs baseline: 4.3691x; 4.3691x over previous
import functools

import jax
import jax.numpy as jnp
from jax import lax
from jax.experimental import pallas as pl
from jax.experimental.pallas import tpu as pltpu
from jax.experimental.pallas import tpu_sc as plsc

N_NODES = 10000
N_EDGES = 320000
FEATS = 128
MSG_W = 32
GW = 128
N_SUBCORES = 16
ZROWS = N_NODES // N_SUBCORES

BN = 1000
BE = 2000

_f32 = jnp.float32


def _mesh():
    return plsc.VectorSubcoreMesh(core_axis_name="core", subcore_axis_name="subcore")


def _sc_gather(tdst, tsrc, dsti, srci):
    grid = N_EDGES // GW

    @functools.partial(
        pl.kernel,
        mesh=_mesh(),
        out_type=(
            jax.ShapeDtypeStruct((N_EDGES, MSG_W), _f32),
            jax.ShapeDtypeStruct((N_EDGES, MSG_W), _f32),
        ),
        compiler_params=pltpu.CompilerParams(use_tc_tiling_on_sc=False),
    )
    def k(td_hbm, ts_hbm, di_hbm, si_hbm, gd_hbm, gs_hbm):
        def body(di_vmem, si_vmem, gd_vmem, gs_vmem):
            pltpu.sync_copy(td_hbm.at[di_vmem.at[0]], gd_vmem)
            pltpu.sync_copy(ts_hbm.at[si_vmem.at[0]], gs_vmem)

        pltpu.emit_pipeline(
            body,
            grid=(grid,),
            in_specs=[
                pl.BlockSpec((1, GW), lambda i: (0, i)),
                pl.BlockSpec((1, GW), lambda i: (0, i)),
            ],
            out_specs=[
                pl.BlockSpec((GW, MSG_W), lambda i: (i, 0)),
                pl.BlockSpec((GW, MSG_W), lambda i: (i, 0)),
            ],
            core_axis_name=("core", "subcore"),
            dimension_semantics=(pltpu.PARALLEL,),
        )(di_hbm, si_hbm, gd_hbm, gs_hbm)

    return k(tdst, tsrc, dsti, srci)


def _sc_scatter_add(msg, dsti, zrows):
    grid = N_EDGES // GW

    @functools.partial(
        pl.kernel,
        mesh=_mesh(),
        out_type=jax.ShapeDtypeStruct((2, N_NODES, MSG_W), _f32),
        scratch_types=[pltpu.VMEM_SHARED((N_NODES, MSG_W), _f32)],
        compiler_params=pltpu.CompilerParams(use_tc_tiling_on_sc=False),
    )
    def k(msg_hbm, di_hbm, z_hbm, out_hbm, acc):
        c = lax.axis_index("core")
        s = lax.axis_index("subcore")
        pltpu.sync_copy(z_hbm, acc.at[pl.ds(s * ZROWS, ZROWS)])
        plsc.subcore_barrier()

        def body(m_vmem, di_vmem):
            pltpu.sync_copy(m_vmem, acc.at[di_vmem.at[0]], add=True)

        pltpu.emit_pipeline(
            body,
            grid=(grid,),
            in_specs=[
                pl.BlockSpec((GW, MSG_W), lambda i: (i, 0)),
                pl.BlockSpec((1, GW), lambda i: (0, i)),
            ],
            out_specs=[],
            core_axis_name=("core", "subcore"),
            dimension_semantics=(pltpu.PARALLEL,),
        )(msg_hbm, di_hbm)

        plsc.subcore_barrier()
        pltpu.sync_copy(
            acc.at[pl.ds(s * ZROWS, ZROWS)],
            out_hbm.at[c].at[pl.ds(s * ZROWS, ZROWS)],
        )

    return k(msg, dsti, zrows)


def _tc_tables(h, pos, wd, ws):
    def body(h_ref, p_ref, wd_ref, ws_ref, td_ref, ts_ref):
        hb = h_ref[...]
        pb = p_ref[...]
        z = jnp.zeros((hb.shape[0], MSG_W - 19), _f32)
        a = jnp.dot(hb, wd_ref[...], preferred_element_type=_f32)
        b = jnp.dot(hb, ws_ref[...], preferred_element_type=_f32)
        td_ref[...] = jnp.concatenate([a, pb, z], axis=1)
        ts_ref[...] = jnp.concatenate([b, pb, z], axis=1)

    grid = (N_NODES // BN,)
    return pl.pallas_call(
        body,
        grid=grid,
        in_specs=[
            pl.BlockSpec((BN, FEATS), lambda i: (i, 0)),
            pl.BlockSpec((BN, 3), lambda i: (i, 0)),
            pl.BlockSpec((FEATS, 16), lambda i: (0, 0)),
            pl.BlockSpec((FEATS, 16), lambda i: (0, 0)),
        ],
        out_specs=[
            pl.BlockSpec((BN, MSG_W), lambda i: (i, 0)),
            pl.BlockSpec((BN, MSG_W), lambda i: (i, 0)),
        ],
        out_shape=(
            jax.ShapeDtypeStruct((N_NODES, MSG_W), _f32),
            jax.ShapeDtypeStruct((N_NODES, MSG_W), _f32),
        ),
    )(h, pos, wd, ws)


def _tc_edge(gd, gs, ea, wd2, wea, be1, e2w, be2, x1w, bx1, x2w, bx2):
    def body(gd_ref, gs_ref, ea_ref, wd2_ref, wea_ref, be1_ref, e2w_ref,
             be2_ref, x1w_ref, bx1_ref, x2w_ref, bx2_ref, out_ref):
        gdb = gd_ref[...]
        gsb = gs_ref[...]
        a = gdb[:, :16]
        b = gsb[:, :16]
        rel = gdb[:, 16:19] - gsb[:, 16:19]
        d2 = jnp.sum(rel * rel, axis=1, keepdims=True)
        pre = (a + b + d2 * wd2_ref[...]
               + jnp.dot(ea_ref[...], wea_ref[...], preferred_element_type=_f32)
               + be1_ref[...])
        m1 = jax.nn.silu(pre)
        m = jax.nn.silu(jnp.dot(m1, e2w_ref[...], preferred_element_type=_f32)
                        + be2_ref[...])
        t = jax.nn.silu(jnp.dot(m, x1w_ref[...], preferred_element_type=_f32)
                        + bx1_ref[...])
        xw = jnp.dot(t, x2w_ref[...], preferred_element_type=_f32) + bx2_ref[...]
        z = jnp.zeros((gdb.shape[0], MSG_W - 19), _f32)
        out_ref[...] = jnp.concatenate([m, rel * xw, z], axis=1)

    grid = (N_EDGES // BE,)
    full = lambda shp: pl.BlockSpec(shp, lambda i: (0, 0))
    return pl.pallas_call(
        body,
        grid=grid,
        in_specs=[
            pl.BlockSpec((BE, MSG_W), lambda i: (i, 0)),
            pl.BlockSpec((BE, MSG_W), lambda i: (i, 0)),
            pl.BlockSpec((BE, 4), lambda i: (i, 0)),
            full((1, 16)), full((4, 16)), full((1, 16)),
            full((16, 16)), full((1, 16)),
            full((16, 16)), full((1, 16)),
            full((16, 1)), full((1, 1)),
        ],
        out_specs=pl.BlockSpec((BE, MSG_W), lambda i: (i, 0)),
        out_shape=jax.ShapeDtypeStruct((N_EDGES, MSG_W), _f32),
    )(gd, gs, ea, wd2, wea, be1, e2w, be2, x1w, bx1, x2w, bx2)


def _tc_update(h, pos, p, h1a, h1b, bh1, h2w, bh2, wd, ws):
    def body(h_ref, pos_ref, p_ref, h1a_ref, h1b_ref, bh1_ref, h2w_ref,
             bh2_ref, wd_ref, ws_ref, hn_ref, td_ref, ts_ref):
        hb = h_ref[...]
        pb = pos_ref[...]
        agg = p_ref[0] + p_ref[1]
        magg = agg[:, :16]
        pagg = agg[:, 16:19]
        hu = jax.nn.silu(
            jnp.dot(hb, h1a_ref[...], preferred_element_type=_f32)
            + jnp.dot(magg, h1b_ref[...], preferred_element_type=_f32)
            + bh1_ref[...])
        hu = jnp.dot(hu, h2w_ref[...], preferred_element_type=_f32) + bh2_ref[...]
        hn = hb + hu
        pn = pb + pagg
        hn_ref[...] = hn
        z = jnp.zeros((hb.shape[0], MSG_W - 19), _f32)
        a = jnp.dot(hn, wd_ref[...], preferred_element_type=_f32)
        b = jnp.dot(hn, ws_ref[...], preferred_element_type=_f32)
        td_ref[...] = jnp.concatenate([a, pn, z], axis=1)
        ts_ref[...] = jnp.concatenate([b, pn, z], axis=1)

    grid = (N_NODES // BN,)
    full = lambda shp: pl.BlockSpec(shp, lambda i: (0, 0))
    return pl.pallas_call(
        body,
        grid=grid,
        in_specs=[
            pl.BlockSpec((BN, FEATS), lambda i: (i, 0)),
            pl.BlockSpec((BN, 3), lambda i: (i, 0)),
            pl.BlockSpec((2, BN, MSG_W), lambda i: (0, i, 0)),
            full((FEATS, FEATS)), full((16, FEATS)), full((1, FEATS)),
            full((FEATS, FEATS)), full((1, FEATS)),
            full((FEATS, 16)), full((FEATS, 16)),
        ],
        out_specs=[
            pl.BlockSpec((BN, FEATS), lambda i: (i, 0)),
            pl.BlockSpec((BN, MSG_W), lambda i: (i, 0)),
            pl.BlockSpec((BN, MSG_W), lambda i: (i, 0)),
        ],
        out_shape=(
            jax.ShapeDtypeStruct((N_NODES, FEATS), _f32),
            jax.ShapeDtypeStruct((N_NODES, MSG_W), _f32),
            jax.ShapeDtypeStruct((N_NODES, MSG_W), _f32),
        ),
    )(h, pos, p, h1a, h1b, bh1, h2w, bh2, wd, ws)


def _tc_final(h, p, h1a, h1b, bh1, h2w, bh2, linw, linb):
    def body(h_ref, p_ref, h1a_ref, h1b_ref, bh1_ref, h2w_ref, bh2_ref,
             linw_ref, linb_ref, out_ref):
        hb = h_ref[...]
        agg = p_ref[0] + p_ref[1]
        magg = agg[:, :16]
        hu = jax.nn.silu(
            jnp.dot(hb, h1a_ref[...], preferred_element_type=_f32)
            + jnp.dot(magg, h1b_ref[...], preferred_element_type=_f32)
            + bh1_ref[...])
        hu = jnp.dot(hu, h2w_ref[...], preferred_element_type=_f32) + bh2_ref[...]
        hn = hb + hu
        out_ref[...] = (jnp.dot(hn, linw_ref[...], preferred_element_type=_f32)
                        + linb_ref[...])

    grid = (N_NODES // BN,)
    full = lambda shp: pl.BlockSpec(shp, lambda i: (0, 0))
    return pl.pallas_call(
        body,
        grid=grid,
        in_specs=[
            pl.BlockSpec((BN, FEATS), lambda i: (i, 0)),
            pl.BlockSpec((2, BN, MSG_W), lambda i: (0, i, 0)),
            full((FEATS, FEATS)), full((16, FEATS)), full((1, FEATS)),
            full((FEATS, FEATS)), full((1, FEATS)),
            full((FEATS, 16)), full((1, 16)),
        ],
        out_specs=pl.BlockSpec((BN, 16), lambda i: (i, 0)),
        out_shape=jax.ShapeDtypeStruct((N_NODES, 16), _f32),
    )(h, p, h1a, h1b, bh1, h2w, bh2, linw, linb)


def _layer_weights(p):
    w1, b1 = p['e1']
    return dict(
        wd=w1[:FEATS], ws=w1[FEATS:2 * FEATS],
        wd2=w1[2 * FEATS:2 * FEATS + 1], wea=w1[2 * FEATS + 1:],
        be1=b1.reshape(1, -1),
        e2w=p['e2'][0], be2=p['e2'][1].reshape(1, -1),
        x1w=p['x1'][0], bx1=p['x1'][1].reshape(1, -1),
        x2w=p['x2'][0], bx2=p['x2'][1].reshape(1, -1),
        h1a=p['h1'][0][:FEATS], h1b=p['h1'][0][FEATS:],
        bh1=p['h1'][1].reshape(1, -1),
        h2w=p['h2'][0], bh2=p['h2'][1].reshape(1, -1),
    )


def kernel(x, edge_index, edge_attr, pos, params):
    src = edge_index[0].reshape(1, N_EDGES)
    dst = edge_index[1].reshape(1, N_EDGES)
    zrows = jnp.zeros((ZROWS, MSG_W), _f32)

    l0 = _layer_weights(params['layers'][0])
    l1 = _layer_weights(params['layers'][1])
    linw, linb = params['lin1']
    linb = linb.reshape(1, -1)

    td, ts = _tc_tables(x, pos, l0['wd'], l0['ws'])
    gd, gs = _sc_gather(td, ts, dst, src)
    msg = _tc_edge(gd, gs, edge_attr, l0['wd2'], l0['wea'], l0['be1'],
                   l0['e2w'], l0['be2'], l0['x1w'], l0['bx1'],
                   l0['x2w'], l0['bx2'])
    p0 = _sc_scatter_add(msg, dst, zrows)

    h1, td2, ts2 = _tc_update(x, pos, p0, l0['h1a'], l0['h1b'], l0['bh1'],
                              l0['h2w'], l0['bh2'], l1['wd'], l1['ws'])

    gd2, gs2 = _sc_gather(td2, ts2, dst, src)
    msg2 = _tc_edge(gd2, gs2, edge_attr, l1['wd2'], l1['wea'], l1['be1'],
                    l1['e2w'], l1['be2'], l1['x1w'], l1['bx1'],
                    l1['x2w'], l1['bx2'])
    p1 = _sc_scatter_add(msg2, dst, zrows)

    return _tc_final(h1, p1, l1['h1a'], l1['h1b'], l1['bh1'],
                     l1['h2w'], l1['bh2'], linw, linb)


# speedup vs baseline: 9.2345x; 2.1136x over previous
import functools

import jax
import jax.numpy as jnp
from jax import lax
from jax.experimental import pallas as pl
from jax.experimental.pallas import tpu as pltpu
from jax.experimental.pallas import tpu_sc as plsc

N_NODES = 10000
N_EDGES = 320000
FEATS = 128
MSG_W = 32
GW = 128
N_SUBCORES = 16
ZROWS = N_NODES // N_SUBCORES

BN = 1000
PACK = 4
BEP = 1000

_f32 = jnp.float32


def _mesh():
    return plsc.VectorSubcoreMesh(core_axis_name="core", subcore_axis_name="subcore")


def _sc_gather(tdst, tsrc, dsti, srci):
    grid = N_EDGES // GW

    @functools.partial(
        pl.kernel,
        mesh=_mesh(),
        out_type=(
            jax.ShapeDtypeStruct((N_EDGES, MSG_W), _f32),
            jax.ShapeDtypeStruct((N_EDGES, MSG_W), _f32),
        ),
        compiler_params=pltpu.CompilerParams(use_tc_tiling_on_sc=False),
    )
    def k(td_hbm, ts_hbm, di_hbm, si_hbm, gd_hbm, gs_hbm):
        def body(di_vmem, si_vmem, gd_vmem, gs_vmem):
            pltpu.sync_copy(td_hbm.at[di_vmem.at[0]], gd_vmem)
            pltpu.sync_copy(ts_hbm.at[si_vmem.at[0]], gs_vmem)

        pltpu.emit_pipeline(
            body,
            grid=(grid,),
            in_specs=[
                pl.BlockSpec((1, GW), lambda i: (0, i)),
                pl.BlockSpec((1, GW), lambda i: (0, i)),
            ],
            out_specs=[
                pl.BlockSpec((GW, MSG_W), lambda i: (i, 0)),
                pl.BlockSpec((GW, MSG_W), lambda i: (i, 0)),
            ],
            core_axis_name=("core", "subcore"),
            dimension_semantics=(pltpu.PARALLEL,),
        )(di_hbm, si_hbm, gd_hbm, gs_hbm)

    return k(tdst, tsrc, dsti, srci)


def _sc_scatter_add(msg, dsti, zrows):
    grid = N_EDGES // GW

    @functools.partial(
        pl.kernel,
        mesh=_mesh(),
        out_type=jax.ShapeDtypeStruct((2, N_NODES, MSG_W), _f32),
        scratch_types=[pltpu.VMEM_SHARED((N_NODES, MSG_W), _f32)],
        compiler_params=pltpu.CompilerParams(use_tc_tiling_on_sc=False),
    )
    def k(msg_hbm, di_hbm, z_hbm, out_hbm, acc):
        c = lax.axis_index("core")
        s = lax.axis_index("subcore")
        pltpu.sync_copy(z_hbm, acc.at[pl.ds(s * ZROWS, ZROWS)])
        plsc.subcore_barrier()

        def body(m_vmem, di_vmem):
            pltpu.sync_copy(m_vmem, acc.at[di_vmem.at[0]], add=True)

        pltpu.emit_pipeline(
            body,
            grid=(grid,),
            in_specs=[
                pl.BlockSpec((GW, MSG_W), lambda i: (i, 0)),
                pl.BlockSpec((1, GW), lambda i: (0, i)),
            ],
            out_specs=[],
            core_axis_name=("core", "subcore"),
            dimension_semantics=(pltpu.PARALLEL,),
        )(msg_hbm, di_hbm)

        plsc.subcore_barrier()
        pltpu.sync_copy(
            acc.at[pl.ds(s * ZROWS, ZROWS)],
            out_hbm.at[c].at[pl.ds(s * ZROWS, ZROWS)],
        )

    return k(msg, dsti, zrows)


def _tc_tables(h, pos, wd, ws):
    def body(h_ref, p_ref, wd_ref, ws_ref, td_ref, ts_ref):
        hb = h_ref[...]
        pb = p_ref[...]
        z = jnp.zeros((hb.shape[0], MSG_W - 19), _f32)
        a = jnp.dot(hb, wd_ref[...], preferred_element_type=_f32)
        b = jnp.dot(hb, ws_ref[...], preferred_element_type=_f32)
        td_ref[...] = jnp.concatenate([a, pb, z], axis=1)
        ts_ref[...] = jnp.concatenate([b, -pb, z], axis=1)

    grid = (N_NODES // BN,)
    return pl.pallas_call(
        body,
        grid=grid,
        in_specs=[
            pl.BlockSpec((BN, FEATS), lambda i: (i, 0)),
            pl.BlockSpec((BN, 3), lambda i: (i, 0)),
            pl.BlockSpec((FEATS, 16), lambda i: (0, 0)),
            pl.BlockSpec((FEATS, 16), lambda i: (0, 0)),
        ],
        out_specs=[
            pl.BlockSpec((BN, MSG_W), lambda i: (i, 0)),
            pl.BlockSpec((BN, MSG_W), lambda i: (i, 0)),
        ],
        out_shape=(
            jax.ShapeDtypeStruct((N_NODES, MSG_W), _f32),
            jax.ShapeDtypeStruct((N_NODES, MSG_W), _f32),
        ),
    )(h, pos, wd, ws)


def _tc_edge(gd, gs, eap, pw):
    def body(gd_ref, gs_ref, ea_ref, sp_ref, weap_ref, e2p_ref, x1p_ref,
             x2p_ref, b1_ref, b2_ref, bx1_ref, bx2_ref, out_ref):
        g = gd_ref[...] + gs_ref[...]
        pre = (g + jnp.dot(g * g, sp_ref[...], preferred_element_type=_f32)
               + jnp.dot(ea_ref[...], weap_ref[...], preferred_element_type=_f32)
               + b1_ref[...])
        m1 = jax.nn.silu(pre)
        m = jax.nn.silu(jnp.dot(m1, e2p_ref[...], preferred_element_type=_f32)
                        + b2_ref[...])
        t = jax.nn.silu(jnp.dot(m, x1p_ref[...], preferred_element_type=_f32)
                        + bx1_ref[...])
        xwb = jnp.dot(t, x2p_ref[...], preferred_element_type=_f32) + bx2_ref[...]
        out_ref[...] = m + g * xwb

    rows = N_EDGES // PACK
    grid = (rows // BEP,)
    full = lambda shp: pl.BlockSpec(shp, lambda i: (0, 0))
    out = pl.pallas_call(
        body,
        grid=grid,
        in_specs=[
            pl.BlockSpec((BEP, 128), lambda i: (i, 0)),
            pl.BlockSpec((BEP, 128), lambda i: (i, 0)),
            pl.BlockSpec((BEP, 16), lambda i: (i, 0)),
            full((128, 128)), full((16, 128)),
            full((128, 128)), full((128, 128)), full((128, 128)),
            full((1, 128)), full((1, 128)), full((1, 128)), full((1, 128)),
        ],
        out_specs=pl.BlockSpec((BEP, 128), lambda i: (i, 0)),
        out_shape=jax.ShapeDtypeStruct((rows, 128), _f32),
    )(gd.reshape(rows, 128), gs.reshape(rows, 128), eap,
      pw['sp'], pw['weap'], pw['e2p'], pw['x1p'], pw['x2p'],
      pw['b1'], pw['b2'], pw['bx1'], pw['bx2'])
    return out.reshape(N_EDGES, MSG_W)


def _tc_update(h, pos, p, h1a, h1b, bh1, h2w, bh2, wd, ws):
    def body(h_ref, pos_ref, p_ref, h1a_ref, h1b_ref, bh1_ref, h2w_ref,
             bh2_ref, wd_ref, ws_ref, hn_ref, td_ref, ts_ref):
        hb = h_ref[...]
        pb = pos_ref[...]
        agg = p_ref[0] + p_ref[1]
        magg = agg[:, :16]
        pagg = agg[:, 16:19]
        hu = jax.nn.silu(
            jnp.dot(hb, h1a_ref[...], preferred_element_type=_f32)
            + jnp.dot(magg, h1b_ref[...], preferred_element_type=_f32)
            + bh1_ref[...])
        hu = jnp.dot(hu, h2w_ref[...], preferred_element_type=_f32) + bh2_ref[...]
        hn = hb + hu
        pn = pb + pagg
        hn_ref[...] = hn
        z = jnp.zeros((hb.shape[0], MSG_W - 19), _f32)
        a = jnp.dot(hn, wd_ref[...], preferred_element_type=_f32)
        b = jnp.dot(hn, ws_ref[...], preferred_element_type=_f32)
        td_ref[...] = jnp.concatenate([a, pn, z], axis=1)
        ts_ref[...] = jnp.concatenate([b, -pn, z], axis=1)

    grid = (N_NODES // BN,)
    full = lambda shp: pl.BlockSpec(shp, lambda i: (0, 0))
    return pl.pallas_call(
        body,
        grid=grid,
        in_specs=[
            pl.BlockSpec((BN, FEATS), lambda i: (i, 0)),
            pl.BlockSpec((BN, 3), lambda i: (i, 0)),
            pl.BlockSpec((2, BN, MSG_W), lambda i: (0, i, 0)),
            full((FEATS, FEATS)), full((16, FEATS)), full((1, FEATS)),
            full((FEATS, FEATS)), full((1, FEATS)),
            full((FEATS, 16)), full((FEATS, 16)),
        ],
        out_specs=[
            pl.BlockSpec((BN, FEATS), lambda i: (i, 0)),
            pl.BlockSpec((BN, MSG_W), lambda i: (i, 0)),
            pl.BlockSpec((BN, MSG_W), lambda i: (i, 0)),
        ],
        out_shape=(
            jax.ShapeDtypeStruct((N_NODES, FEATS), _f32),
            jax.ShapeDtypeStruct((N_NODES, MSG_W), _f32),
            jax.ShapeDtypeStruct((N_NODES, MSG_W), _f32),
        ),
    )(h, pos, p, h1a, h1b, bh1, h2w, bh2, wd, ws)


def _tc_final(h, p, h1a, h1b, bh1, h2w, bh2, linw, linb):
    def body(h_ref, p_ref, h1a_ref, h1b_ref, bh1_ref, h2w_ref, bh2_ref,
             linw_ref, linb_ref, out_ref):
        hb = h_ref[...]
        agg = p_ref[0] + p_ref[1]
        magg = agg[:, :16]
        hu = jax.nn.silu(
            jnp.dot(hb, h1a_ref[...], preferred_element_type=_f32)
            + jnp.dot(magg, h1b_ref[...], preferred_element_type=_f32)
            + bh1_ref[...])
        hu = jnp.dot(hu, h2w_ref[...], preferred_element_type=_f32) + bh2_ref[...]
        hn = hb + hu
        out_ref[...] = (jnp.dot(hn, linw_ref[...], preferred_element_type=_f32)
                        + linb_ref[...])

    grid = (N_NODES // BN,)
    full = lambda shp: pl.BlockSpec(shp, lambda i: (0, 0))
    return pl.pallas_call(
        body,
        grid=grid,
        in_specs=[
            pl.BlockSpec((BN, FEATS), lambda i: (i, 0)),
            pl.BlockSpec((2, BN, MSG_W), lambda i: (0, i, 0)),
            full((FEATS, FEATS)), full((16, FEATS)), full((1, FEATS)),
            full((FEATS, FEATS)), full((1, FEATS)),
            full((FEATS, 16)), full((1, 16)),
        ],
        out_specs=pl.BlockSpec((BN, 16), lambda i: (i, 0)),
        out_shape=jax.ShapeDtypeStruct((N_NODES, 16), _f32),
    )(h, p, h1a, h1b, bh1, h2w, bh2, linw, linb)


def _layer_weights(p):
    w1, b1 = p['e1']
    return dict(
        wd=w1[:FEATS], ws=w1[FEATS:2 * FEATS],
        wd2=w1[2 * FEATS:2 * FEATS + 1], wea=w1[2 * FEATS + 1:],
        be1=b1.reshape(1, -1),
        e2w=p['e2'][0], be2=p['e2'][1].reshape(1, -1),
        x1w=p['x1'][0], bx1=p['x1'][1].reshape(1, -1),
        x2w=p['x2'][0], bx2=p['x2'][1].reshape(1, -1),
        h1a=p['h1'][0][:FEATS], h1b=p['h1'][0][FEATS:],
        bh1=p['h1'][1].reshape(1, -1),
        h2w=p['h2'][0], bh2=p['h2'][1].reshape(1, -1),
    )


def _pack_edge_weights(l):
    z128 = jnp.zeros((128, 128), _f32)
    sp = z128
    e2p = z128
    x1p = z128
    x2p = z128
    weap = jnp.zeros((16, 128), _f32)
    b1 = jnp.zeros((1, 128), _f32)
    b2 = jnp.zeros((1, 128), _f32)
    bx1 = jnp.zeros((1, 128), _f32)
    bx2 = jnp.zeros((1, 128), _f32)
    for i in range(PACK):
        o = 32 * i
        hs = slice(o, o + 16)
        rs = slice(o + 16, o + 19)
        sp = sp.at[rs, hs].set(jnp.broadcast_to(l['wd2'], (3, 16)))
        e2p = e2p.at[hs, hs].set(l['e2w'])
        x1p = x1p.at[hs, hs].set(l['x1w'])
        x2p = x2p.at[hs, rs].set(jnp.broadcast_to(l['x2w'], (16, 3)))
        weap = weap.at[4 * i:4 * i + 4, hs].set(l['wea'])
        b1 = b1.at[0, hs].set(l['be1'][0])
        b2 = b2.at[0, hs].set(l['be2'][0])
        bx1 = bx1.at[0, hs].set(l['bx1'][0])
        bx2 = bx2.at[0, rs].set(l['bx2'][0, 0])
    return dict(sp=sp, e2p=e2p, x1p=x1p, x2p=x2p, weap=weap,
                b1=b1, b2=b2, bx1=bx1, bx2=bx2)


def kernel(x, edge_index, edge_attr, pos, params):
    src = edge_index[0].reshape(1, N_EDGES)
    dst = edge_index[1].reshape(1, N_EDGES)
    zrows = jnp.zeros((ZROWS, MSG_W), _f32)

    l0 = _layer_weights(params['layers'][0])
    l1 = _layer_weights(params['layers'][1])
    pw0 = _pack_edge_weights(l0)
    pw1 = _pack_edge_weights(l1)
    eap = edge_attr.reshape(N_EDGES // PACK, 4 * PACK)
    linw, linb = params['lin1']
    linb = linb.reshape(1, -1)

    td, ts = _tc_tables(x, pos, l0['wd'], l0['ws'])
    gd, gs = _sc_gather(td, ts, dst, src)
    msg = _tc_edge(gd, gs, eap, pw0)
    p0 = _sc_scatter_add(msg, dst, zrows)

    h1, td2, ts2 = _tc_update(x, pos, p0, l0['h1a'], l0['h1b'], l0['bh1'],
                              l0['h2w'], l0['bh2'], l1['wd'], l1['ws'])

    gd2, gs2 = _sc_gather(td2, ts2, dst, src)
    msg2 = _tc_edge(gd2, gs2, eap, pw1)
    p1 = _sc_scatter_add(msg2, dst, zrows)

    return _tc_final(h1, p1, l1['h1a'], l1['h1b'], l1['bh1'],
                     l1['h2w'], l1['bh2'], linw, linb)


# speedup vs baseline: 10.8578x; 1.1758x over previous
import functools

import jax
import jax.numpy as jnp
from jax import lax
from jax.experimental import pallas as pl
from jax.experimental.pallas import tpu as pltpu
from jax.experimental.pallas import tpu_sc as plsc

N_NODES = 10000
N_EDGES = 320000
FEATS = 128
MSG_W = 32
GW = 128
N_SUBCORES = 16
ZROWS = N_NODES // N_SUBCORES

BN = 1000
PACK = 4
BEP = 1000

_f32 = jnp.float32


def _mesh():
    return plsc.VectorSubcoreMesh(core_axis_name="core", subcore_axis_name="subcore")


def _sc_gather(tdst, tsrc, dsti, srci2):
    grid = N_EDGES // GW
    srows = N_NODES // N_SUBCORES

    @functools.partial(
        pl.kernel,
        mesh=_mesh(),
        out_type=jax.ShapeDtypeStruct((N_EDGES, MSG_W), _f32),
        scratch_types=[pltpu.VMEM_SHARED((2 * N_NODES, MSG_W), _f32)],
        compiler_params=pltpu.CompilerParams(use_tc_tiling_on_sc=False),
    )
    def k(td_hbm, ts_hbm, di_hbm, si_hbm, g_hbm, tab):
        s = lax.axis_index("subcore")
        pltpu.sync_copy(td_hbm.at[pl.ds(s * srows, srows)],
                        tab.at[pl.ds(s * srows, srows)])
        pltpu.sync_copy(ts_hbm.at[pl.ds(s * srows, srows)],
                        tab.at[pl.ds(N_NODES + s * srows, srows)])
        plsc.subcore_barrier()

        def body(di_vmem, si_vmem, g_vmem):
            pltpu.sync_copy(tab.at[di_vmem.at[0]], g_vmem)
            pltpu.sync_copy(tab.at[si_vmem.at[0]], g_vmem, add=True)

        pltpu.emit_pipeline(
            body,
            grid=(grid,),
            in_specs=[
                pl.BlockSpec((1, GW), lambda i: (0, i)),
                pl.BlockSpec((1, GW), lambda i: (0, i)),
            ],
            out_specs=[
                pl.BlockSpec((GW, MSG_W), lambda i: (i, 0)),
            ],
            core_axis_name=("core", "subcore"),
            dimension_semantics=(pltpu.PARALLEL,),
        )(di_hbm, si_hbm, g_hbm)

    return k(tdst, tsrc, dsti, srci2)


def _sc_scatter_add(msg, dsti, zrows):
    grid = N_EDGES // GW

    @functools.partial(
        pl.kernel,
        mesh=_mesh(),
        out_type=jax.ShapeDtypeStruct((2, N_NODES, MSG_W), _f32),
        scratch_types=[pltpu.VMEM_SHARED((N_NODES, MSG_W), _f32)],
        compiler_params=pltpu.CompilerParams(use_tc_tiling_on_sc=False),
    )
    def k(msg_hbm, di_hbm, z_hbm, out_hbm, acc):
        c = lax.axis_index("core")
        s = lax.axis_index("subcore")
        pltpu.sync_copy(z_hbm, acc.at[pl.ds(s * ZROWS, ZROWS)])
        plsc.subcore_barrier()

        def body(m_vmem, di_vmem):
            pltpu.sync_copy(m_vmem, acc.at[di_vmem.at[0]], add=True)

        pltpu.emit_pipeline(
            body,
            grid=(grid,),
            in_specs=[
                pl.BlockSpec((GW, MSG_W), lambda i: (i, 0)),
                pl.BlockSpec((1, GW), lambda i: (0, i)),
            ],
            out_specs=[],
            core_axis_name=("core", "subcore"),
            dimension_semantics=(pltpu.PARALLEL,),
        )(msg_hbm, di_hbm)

        plsc.subcore_barrier()
        pltpu.sync_copy(
            acc.at[pl.ds(s * ZROWS, ZROWS)],
            out_hbm.at[c].at[pl.ds(s * ZROWS, ZROWS)],
        )

    return k(msg, dsti, zrows)


def _tc_tables(h, pos, wd, ws):
    def body(h_ref, p_ref, wd_ref, ws_ref, td_ref, ts_ref):
        hb = h_ref[...]
        pb = p_ref[...]
        z = jnp.zeros((hb.shape[0], MSG_W - 19), _f32)
        a = jnp.dot(hb, wd_ref[...], preferred_element_type=_f32)
        b = jnp.dot(hb, ws_ref[...], preferred_element_type=_f32)
        td_ref[...] = jnp.concatenate([a, pb, z], axis=1)
        ts_ref[...] = jnp.concatenate([b, -pb, z], axis=1)

    grid = (N_NODES // BN,)
    return pl.pallas_call(
        body,
        grid=grid,
        in_specs=[
            pl.BlockSpec((BN, FEATS), lambda i: (i, 0)),
            pl.BlockSpec((BN, 3), lambda i: (i, 0)),
            pl.BlockSpec((FEATS, 16), lambda i: (0, 0)),
            pl.BlockSpec((FEATS, 16), lambda i: (0, 0)),
        ],
        out_specs=[
            pl.BlockSpec((BN, MSG_W), lambda i: (i, 0)),
            pl.BlockSpec((BN, MSG_W), lambda i: (i, 0)),
        ],
        out_shape=(
            jax.ShapeDtypeStruct((N_NODES, MSG_W), _f32),
            jax.ShapeDtypeStruct((N_NODES, MSG_W), _f32),
        ),
    )(h, pos, wd, ws)


def _tc_edge(gin, eap, pw):
    def body(g_ref, ea_ref, sp_ref, weap_ref, e2p_ref, x1p_ref,
             x2p_ref, b1_ref, b2_ref, bx1_ref, bx2_ref, out_ref):
        g = g_ref[...]
        pre = (g + jnp.dot(g * g, sp_ref[...], preferred_element_type=_f32)
               + jnp.dot(ea_ref[...], weap_ref[...], preferred_element_type=_f32)
               + b1_ref[...])
        m1 = jax.nn.silu(pre)
        m = jax.nn.silu(jnp.dot(m1, e2p_ref[...], preferred_element_type=_f32)
                        + b2_ref[...])
        t = jax.nn.silu(jnp.dot(m, x1p_ref[...], preferred_element_type=_f32)
                        + bx1_ref[...])
        xwb = jnp.dot(t, x2p_ref[...], preferred_element_type=_f32) + bx2_ref[...]
        out_ref[...] = m + g * xwb

    rows = N_EDGES // PACK
    grid = (rows // BEP,)
    full = lambda shp: pl.BlockSpec(shp, lambda i: (0, 0))
    out = pl.pallas_call(
        body,
        grid=grid,
        in_specs=[
            pl.BlockSpec((BEP, 128), lambda i: (i, 0)),
            pl.BlockSpec((BEP, 16), lambda i: (i, 0)),
            full((128, 128)), full((16, 128)),
            full((128, 128)), full((128, 128)), full((128, 128)),
            full((1, 128)), full((1, 128)), full((1, 128)), full((1, 128)),
        ],
        out_specs=pl.BlockSpec((BEP, 128), lambda i: (i, 0)),
        out_shape=jax.ShapeDtypeStruct((rows, 128), _f32),
    )(gin.reshape(rows, 128), eap,
      pw['sp'], pw['weap'], pw['e2p'], pw['x1p'], pw['x2p'],
      pw['b1'], pw['b2'], pw['bx1'], pw['bx2'])
    return out.reshape(N_EDGES, MSG_W)


def _tc_update(h, pos, p, h1a, h1b, bh1, h2w, bh2, wd, ws):
    def body(h_ref, pos_ref, p_ref, h1a_ref, h1b_ref, bh1_ref, h2w_ref,
             bh2_ref, wd_ref, ws_ref, hn_ref, td_ref, ts_ref):
        hb = h_ref[...]
        pb = pos_ref[...]
        agg = p_ref[0] + p_ref[1]
        magg = agg[:, :16]
        pagg = agg[:, 16:19]
        hu = jax.nn.silu(
            jnp.dot(hb, h1a_ref[...], preferred_element_type=_f32)
            + jnp.dot(magg, h1b_ref[...], preferred_element_type=_f32)
            + bh1_ref[...])
        hu = jnp.dot(hu, h2w_ref[...], preferred_element_type=_f32) + bh2_ref[...]
        hn = hb + hu
        pn = pb + pagg
        hn_ref[...] = hn
        z = jnp.zeros((hb.shape[0], MSG_W - 19), _f32)
        a = jnp.dot(hn, wd_ref[...], preferred_element_type=_f32)
        b = jnp.dot(hn, ws_ref[...], preferred_element_type=_f32)
        td_ref[...] = jnp.concatenate([a, pn, z], axis=1)
        ts_ref[...] = jnp.concatenate([b, -pn, z], axis=1)

    grid = (N_NODES // BN,)
    full = lambda shp: pl.BlockSpec(shp, lambda i: (0, 0))
    return pl.pallas_call(
        body,
        grid=grid,
        in_specs=[
            pl.BlockSpec((BN, FEATS), lambda i: (i, 0)),
            pl.BlockSpec((BN, 3), lambda i: (i, 0)),
            pl.BlockSpec((2, BN, MSG_W), lambda i: (0, i, 0)),
            full((FEATS, FEATS)), full((16, FEATS)), full((1, FEATS)),
            full((FEATS, FEATS)), full((1, FEATS)),
            full((FEATS, 16)), full((FEATS, 16)),
        ],
        out_specs=[
            pl.BlockSpec((BN, FEATS), lambda i: (i, 0)),
            pl.BlockSpec((BN, MSG_W), lambda i: (i, 0)),
            pl.BlockSpec((BN, MSG_W), lambda i: (i, 0)),
        ],
        out_shape=(
            jax.ShapeDtypeStruct((N_NODES, FEATS), _f32),
            jax.ShapeDtypeStruct((N_NODES, MSG_W), _f32),
            jax.ShapeDtypeStruct((N_NODES, MSG_W), _f32),
        ),
    )(h, pos, p, h1a, h1b, bh1, h2w, bh2, wd, ws)


def _tc_final(h, p, h1a, h1b, bh1, h2w, bh2, linw, linb):
    def body(h_ref, p_ref, h1a_ref, h1b_ref, bh1_ref, h2w_ref, bh2_ref,
             linw_ref, linb_ref, out_ref):
        hb = h_ref[...]
        agg = p_ref[0] + p_ref[1]
        magg = agg[:, :16]
        hu = jax.nn.silu(
            jnp.dot(hb, h1a_ref[...], preferred_element_type=_f32)
            + jnp.dot(magg, h1b_ref[...], preferred_element_type=_f32)
            + bh1_ref[...])
        hu = jnp.dot(hu, h2w_ref[...], preferred_element_type=_f32) + bh2_ref[...]
        hn = hb + hu
        out_ref[...] = (jnp.dot(hn, linw_ref[...], preferred_element_type=_f32)
                        + linb_ref[...])

    grid = (N_NODES // BN,)
    full = lambda shp: pl.BlockSpec(shp, lambda i: (0, 0))
    return pl.pallas_call(
        body,
        grid=grid,
        in_specs=[
            pl.BlockSpec((BN, FEATS), lambda i: (i, 0)),
            pl.BlockSpec((2, BN, MSG_W), lambda i: (0, i, 0)),
            full((FEATS, FEATS)), full((16, FEATS)), full((1, FEATS)),
            full((FEATS, FEATS)), full((1, FEATS)),
            full((FEATS, 16)), full((1, 16)),
        ],
        out_specs=pl.BlockSpec((BN, 16), lambda i: (i, 0)),
        out_shape=jax.ShapeDtypeStruct((N_NODES, 16), _f32),
    )(h, p, h1a, h1b, bh1, h2w, bh2, linw, linb)


def _layer_weights(p):
    w1, b1 = p['e1']
    return dict(
        wd=w1[:FEATS], ws=w1[FEATS:2 * FEATS],
        wd2=w1[2 * FEATS:2 * FEATS + 1], wea=w1[2 * FEATS + 1:],
        be1=b1.reshape(1, -1),
        e2w=p['e2'][0], be2=p['e2'][1].reshape(1, -1),
        x1w=p['x1'][0], bx1=p['x1'][1].reshape(1, -1),
        x2w=p['x2'][0], bx2=p['x2'][1].reshape(1, -1),
        h1a=p['h1'][0][:FEATS], h1b=p['h1'][0][FEATS:],
        bh1=p['h1'][1].reshape(1, -1),
        h2w=p['h2'][0], bh2=p['h2'][1].reshape(1, -1),
    )


def _pack_edge_weights(l):
    z128 = jnp.zeros((128, 128), _f32)
    sp = z128
    e2p = z128
    x1p = z128
    x2p = z128
    weap = jnp.zeros((16, 128), _f32)
    b1 = jnp.zeros((1, 128), _f32)
    b2 = jnp.zeros((1, 128), _f32)
    bx1 = jnp.zeros((1, 128), _f32)
    bx2 = jnp.zeros((1, 128), _f32)
    for i in range(PACK):
        o = 32 * i
        hs = slice(o, o + 16)
        rs = slice(o + 16, o + 19)
        sp = sp.at[rs, hs].set(jnp.broadcast_to(l['wd2'], (3, 16)))
        e2p = e2p.at[hs, hs].set(l['e2w'])
        x1p = x1p.at[hs, hs].set(l['x1w'])
        x2p = x2p.at[hs, rs].set(jnp.broadcast_to(l['x2w'], (16, 3)))
        weap = weap.at[4 * i:4 * i + 4, hs].set(l['wea'])
        b1 = b1.at[0, hs].set(l['be1'][0])
        b2 = b2.at[0, hs].set(l['be2'][0])
        bx1 = bx1.at[0, hs].set(l['bx1'][0])
        bx2 = bx2.at[0, rs].set(l['bx2'][0, 0])
    return dict(sp=sp, e2p=e2p, x1p=x1p, x2p=x2p, weap=weap,
                b1=b1, b2=b2, bx1=bx1, bx2=bx2)


def kernel(x, edge_index, edge_attr, pos, params):
    src2 = (edge_index[0] + N_NODES).reshape(1, N_EDGES)
    dst = edge_index[1].reshape(1, N_EDGES)
    zrows = jnp.zeros((ZROWS, MSG_W), _f32)

    l0 = _layer_weights(params['layers'][0])
    l1 = _layer_weights(params['layers'][1])
    pw0 = _pack_edge_weights(l0)
    pw1 = _pack_edge_weights(l1)
    eap = edge_attr.reshape(N_EDGES // PACK, 4 * PACK)
    linw, linb = params['lin1']
    linb = linb.reshape(1, -1)

    td, ts = _tc_tables(x, pos, l0['wd'], l0['ws'])
    g = _sc_gather(td, ts, dst, src2)
    msg = _tc_edge(g, eap, pw0)
    p0 = _sc_scatter_add(msg, dst, zrows)

    h1, td2, ts2 = _tc_update(x, pos, p0, l0['h1a'], l0['h1b'], l0['bh1'],
                              l0['h2w'], l0['bh2'], l1['wd'], l1['ws'])

    g2 = _sc_gather(td2, ts2, dst, src2)
    msg2 = _tc_edge(g2, eap, pw1)
    p1 = _sc_scatter_add(msg2, dst, zrows)

    return _tc_final(h1, p1, l1['h1a'], l1['h1b'], l1['bh1'],
                     l1['h2w'], l1['bh2'], linw, linb)
